# baseline (speedup 1.0000x reference)
import functools

import jax
import jax.numpy as jnp
from jax import lax
from jax.experimental import pallas as pl
from jax.experimental.pallas import tpu as pltpu
from jax.experimental.pallas import tpu_sc as plsc

B = 16384
D = 32
NC = 2
NS = 16
L = 16
NW = NC * NS
CHUNK = B // NW
GSZ = 128
NG = CHUNK // GSZ

_mesh = plsc.VectorSubcoreMesh(core_axis_name="c", subcore_axis_name="s")


@functools.partial(
    pl.kernel,
    mesh=_mesh,
    out_type=jax.ShapeDtypeStruct((B,), jnp.float32),
    compiler_params=pltpu.CompilerParams(needs_layout_passes=False,
                                         use_tc_tiling_on_sc=False),
    scratch_types=[
        pltpu.VMEM((CHUNK,), jnp.int32),
        pltpu.VMEM((CHUNK,), jnp.int32),
        pltpu.VMEM((CHUNK, D), jnp.float32),
        pltpu.VMEM((CHUNK, D), jnp.float32),
        pltpu.VMEM((CHUNK,), jnp.float32),
        pltpu.SemaphoreType.DMA,
    ],
)
def _gmf(uidx_hbm, iidx_hbm, utab_hbm, itab_hbm, out_hbm,
         uidx_v, iidx_v, urows_v, irows_v, out_v, sem):
    wid = lax.axis_index("s") * NC + lax.axis_index("c")
    base = wid * CHUNK

    pltpu.sync_copy(uidx_hbm.at[pl.ds(base, CHUNK)], uidx_v)
    pltpu.sync_copy(iidx_hbm.at[pl.ds(base, CHUNK)], iidx_v)

    copies = []
    for j in range(NG):
        sl = pl.ds(j * GSZ, GSZ)
        copies.append(pltpu.async_copy(utab_hbm.at[uidx_v.at[sl]],
                                       urows_v.at[sl], sem))
        copies.append(pltpu.async_copy(itab_hbm.at[iidx_v.at[sl]],
                                       irows_v.at[sl], sem))
    for c in copies:
        c.wait()

    def group(g, carry):
        rows = g * L + lax.iota(jnp.int32, L)
        acc = jnp.zeros((L,), jnp.float32)
        for d in range(D):
            cols = jnp.full((L,), d, jnp.int32)
            u = plsc.load_gather(urows_v, [rows, cols])
            w = plsc.load_gather(irows_v, [rows, cols])
            acc = acc + u * w
        out_v[pl.ds(g * L, L)] = acc
        return carry

    lax.fori_loop(0, CHUNK // L, group, 0)

    pltpu.sync_copy(out_v, out_hbm.at[pl.ds(base, CHUNK)])


def kernel(user_input, item_input, user_table, item_table):
    return _gmf(user_input.astype(jnp.int32), item_input.astype(jnp.int32),
                user_table, item_table)
